# Initial kernel scaffold; baseline (speedup 1.0000x reference)
import jax
import jax.numpy as jnp
from jax.experimental import pallas as pl


def kernel(gp_x, gp_edge_index, sp_x, sp_edge_index, params):
    raise NotImplementedError("write your pallas kernel here")


# speedup vs baseline: 10.3777x; 10.3777x over previous
import math

import jax
import jax.numpy as jnp
from jax import lax
from jax.experimental import pallas as pl
from jax.experimental.pallas import tpu as pltpu
from jax.experimental.pallas import tpu_sc as plsc

N_NODES = 50000
N_EDGES = 800000
NP = 51200
NR = NP // 128
E_TOT = N_EDGES + N_NODES
EW = 6656
EP = EW * 128
HID = 128
NCH = 4
CW = 32
N_TILES = 16
WPT = EW // N_TILES
RPT = NP // N_TILES

BN = 2048
BNR = BN // 128

def _vec_mesh():
  return plsc.VectorSubcoreMesh(core_axis_name="c", subcore_axis_name="s",
                                num_cores=2, num_subcores=16)


def _sc_wide_body(table, srcw, dstw, out, accum, zbuf, sidx, didx, rows, sems):
  c = lax.axis_index("c")
  s = lax.axis_index("s")
  w0 = s * WPT
  r0 = s * RPT

  @pl.loop(0, RPT // 8)
  def _(r):
    @pl.loop(0, CW, step=16)
    def _(col):
      zbuf[r, pl.ds(col, 16)] = jnp.zeros((16,), jnp.float32)

  for job in range(4):
    jid = c * 4 + job
    for z in range(8):
      pltpu.sync_copy(zbuf, accum.at[pl.ds(r0 + z * (RPT // 8), RPT // 8)])
    plsc.subcore_barrier()

    @pl.loop(0, WPT)
    def _(w):
      pltpu.sync_copy(srcw.at[c].at[w0 + w], sidx)
      pltpu.sync_copy(dstw.at[c].at[w0 + w], didx)
      pltpu.async_copy(table.at[jid].at[sidx], rows, sems.at[0]).wait()
      pltpu.async_copy(rows, accum.at[didx], sems.at[1], add=True).wait()

    plsc.subcore_barrier()
    pltpu.sync_copy(accum.at[pl.ds(r0, RPT)], out.at[jid].at[pl.ds(r0, RPT)])
    plsc.subcore_barrier()


@jax.jit
def _sc_wide(table, srcw, dstw):
  kern = pl.kernel(
      _sc_wide_body,
      out_type=jax.ShapeDtypeStruct((8, NP, CW), jnp.float32),
      mesh=_vec_mesh(),
      compiler_params=pltpu.CompilerParams(use_tc_tiling_on_sc=False),
      scratch_types=[
          pltpu.VMEM_SHARED((NP, CW), jnp.float32),
          pltpu.VMEM((RPT // 8, CW), jnp.float32),
          pltpu.VMEM((128,), jnp.int32),
          pltpu.VMEM((128,), jnp.int32),
          pltpu.VMEM((128, CW), jnp.float32),
          pltpu.SemaphoreType.DMA((2,)),
      ],
  )
  return kern(table, srcw, dstw)


def _sc_deg_body(alive, srcw, dstw, out, accum, zbuf, sidx, didx, vals, sems):
  b = lax.axis_index("c")
  s = lax.axis_index("s")
  w0 = s * WPT
  r0 = s * RPT

  @pl.loop(0, RPT // 8)
  def _(r):
    zbuf[r, pl.ds(0, 16)] = jnp.zeros((16,), jnp.float32)

  for z in range(8):
    pltpu.sync_copy(zbuf, accum.at[pl.ds(r0 + z * (RPT // 8), RPT // 8)])
  plsc.subcore_barrier()

  @pl.loop(0, WPT)
  def _(w):
    pltpu.sync_copy(srcw.at[b].at[w0 + w], sidx)
    pltpu.sync_copy(dstw.at[b].at[w0 + w], didx)
    pltpu.async_copy(alive.at[b].at[sidx], vals, sems.at[0]).wait()
    pltpu.async_copy(vals, accum.at[didx], sems.at[1], add=True).wait()

  plsc.subcore_barrier()
  pltpu.sync_copy(accum.at[pl.ds(r0, RPT)], out.at[b].at[pl.ds(r0, RPT)])
  plsc.subcore_barrier()


@jax.jit
def _sc_deg(alive, srcw, dstw):
  kern = pl.kernel(
      _sc_deg_body,
      out_type=jax.ShapeDtypeStruct((2, NP, 16), jnp.float32),
      mesh=_vec_mesh(),
      compiler_params=pltpu.CompilerParams(use_tc_tiling_on_sc=False),
      scratch_types=[
          pltpu.VMEM_SHARED((NP, 16), jnp.float32),
          pltpu.VMEM((RPT // 8, 16), jnp.float32),
          pltpu.VMEM((128,), jnp.int32),
          pltpu.VMEM((128,), jnp.int32),
          pltpu.VMEM((128, 16), jnp.float32),
          pltpu.SemaphoreType.DMA((2,)),
      ],
  )
  return kern(alive, srcw, dstw)


def _k1_body(xm_ref, coef_ref, deg_ref, w_ref, xw_ref, y_ref):
  xm = jnp.concatenate([xm_ref[0, i] for i in range(NCH)], axis=-1)
  coef = coef_ref[0]
  xw = jnp.dot((xm * coef).astype(jnp.bfloat16), w_ref[0].astype(jnp.bfloat16),
               preferred_element_type=jnp.float32)
  xw_ref[0] = xw
  scale = 1.0 / jnp.sqrt(deg_ref[0] + 1.0)
  y = xw * scale
  for i in range(NCH):
    y_ref[0, i] = y[:, i * CW:(i + 1) * CW]


def _k1(xm, coef, deg, W):
  return pl.pallas_call(
      _k1_body,
      grid=(2, NP // BN),
      in_specs=[
          pl.BlockSpec((1, NCH, BN, CW), lambda b, i: (b, 0, i, 0)),
          pl.BlockSpec((1, BN, 1), lambda b, i: (b, i, 0)),
          pl.BlockSpec((1, BN, 1), lambda b, i: (b, i, 0)),
          pl.BlockSpec((1, HID, HID), lambda b, i: (b, 0, 0)),
      ],
      out_specs=[
          pl.BlockSpec((1, BN, HID), lambda b, i: (b, i, 0)),
          pl.BlockSpec((1, NCH, BN, CW), lambda b, i: (b, 0, i, 0)),
      ],
      out_shape=[
          jax.ShapeDtypeStruct((2, NP, HID), jnp.float32),
          jax.ShapeDtypeStruct((2, NCH, NP, CW), jnp.float32),
      ],
  )(xm, coef, deg, W)


def _k3_body(msg_ref, xw_ref, deg_ref, alive_ref, b_ref, out_ref):
  scale = 1.0 / jnp.sqrt(deg_ref[0] + 1.0)
  alive = alive_ref[0]
  msg = jnp.concatenate([msg_ref[0, i] for i in range(NCH)], axis=-1)
  xw = xw_ref[0]
  out = jnp.maximum(msg * scale + xw * (scale * scale) + b_ref[0], 0.0) * alive
  for i in range(NCH):
    out_ref[0, i] = out[:, i * CW:(i + 1) * CW]


def _k3(msg, xw, deg, alive, b):
  return pl.pallas_call(
      _k3_body,
      grid=(2, NP // BN),
      in_specs=[
          pl.BlockSpec((1, NCH, BN, CW), lambda b_, i: (b_, 0, i, 0)),
          pl.BlockSpec((1, BN, HID), lambda b_, i: (b_, i, 0)),
          pl.BlockSpec((1, BN, 1), lambda b_, i: (b_, i, 0)),
          pl.BlockSpec((1, BN, 1), lambda b_, i: (b_, i, 0)),
          pl.BlockSpec((1, 1, HID), lambda b_, i: (b_, 0, 0)),
      ],
      out_specs=pl.BlockSpec((1, NCH, BN, CW), lambda b_, i: (b_, 0, i, 0)),
      out_shape=jax.ShapeDtypeStruct((2, NCH, NP, CW), jnp.float32),
  )(msg, xw, deg, alive, b)


def _k4_body(agg_ref, xm_ref, relw_ref, rootw_ref, relb_ref, alive_ref,
             out_ref, tanh_ref):
  b = pl.program_id(0)
  def b16(v):
    return v.astype(jnp.bfloat16).astype(jnp.float32)

  srow = jnp.zeros((BN, 1), jnp.float32)
  for i in range(NCH):
    srow += jnp.sum(b16(agg_ref[0, i]) * b16(relw_ref[0, :, i * CW:(i + 1) * CW]),
                    axis=-1, keepdims=True)
    srow += jnp.sum(b16(xm_ref[0, i]) * b16(rootw_ref[0, :, i * CW:(i + 1) * CW]),
                    axis=-1, keepdims=True)
  pre = srow + relb_ref[b]
  alive = alive_ref[0]
  out_ref[0] = jnp.where(alive > 0.0, pre, -1e30)
  tanh_ref[0] = jnp.tanh(pre)


def _k4(agg, xm, relw, rootw, relb, alive):
  return pl.pallas_call(
      _k4_body,
      grid=(2, NP // BN),
      in_specs=[
          pl.BlockSpec((1, NCH, BN, CW), lambda b, i: (b, 0, i, 0)),
          pl.BlockSpec((1, NCH, BN, CW), lambda b, i: (b, 0, i, 0)),
          pl.BlockSpec((1, 1, HID), lambda b, i: (b, 0, 0)),
          pl.BlockSpec((1, 1, HID), lambda b, i: (b, 0, 0)),
          pl.BlockSpec(memory_space=pltpu.SMEM),
          pl.BlockSpec((1, BN, 1), lambda b, i: (b, i, 0)),
      ],
      out_specs=[
          pl.BlockSpec((1, BN, 1), lambda b, i: (b, i, 0)),
          pl.BlockSpec((1, BN, 1), lambda b, i: (b, i, 0)),
      ],
      out_shape=[
          jax.ShapeDtypeStruct((2, NP, 1), jnp.float32),
          jax.ShapeDtypeStruct((2, NP, 1), jnp.float32),
      ],
  )(agg, xm, relw, rootw, relb, alive)


def _k5_body(k_arr, msc_ref, tanh_ref, coef_ref, alive_ref):
  idx = (lax.broadcasted_iota(jnp.int32, (NR, 128), 0) * 128
         + lax.broadcasted_iota(jnp.int32, (NR, 128), 1))
  for b in range(2):
    k = k_arr[b]
    score = msc_ref[b]
    u = lax.bitcast_convert_type(score, jnp.int32)
    key = jnp.where(u >= 0, u, u ^ jnp.int32(0x7FFFFFFF))

    def cnt_ge(t):
      return jnp.sum((key >= t).astype(jnp.int32))

    t0 = jnp.where(cnt_ge(jnp.int32(0)) >= k, jnp.int32(0),
                   jnp.int32(-2147483648))

    def bit_step(i, t):
      t_try = jnp.bitwise_or(t, jnp.left_shift(jnp.int32(1), 30 - i))
      return jnp.where(cnt_ge(t_try) >= k, t_try, t)

    tau = lax.fori_loop(0, 31, bit_step, t0)
    n_gt = jnp.sum((key > tau).astype(jnp.int32))
    r = k - n_gt

    def tie_step(_, lohi):
      lo, hi = lohi
      mid = (lo + hi) // 2
      c = jnp.sum(((key == tau) & (idx < mid)).astype(jnp.int32))
      good = c >= r
      return jnp.where(good, lo, mid + 1), jnp.where(good, mid, hi)

    _, cut = lax.fori_loop(0, 17, tie_step, (jnp.int32(0), jnp.int32(NP)))
    sel = ((key > tau) | ((key == tau) & (idx < cut))).astype(jnp.float32)
    coef_ref[b] = tanh_ref[b] * sel
    alive_ref[b] = sel


def _k5(ks, msc, tanhv):
  return pl.pallas_call(
      _k5_body,
      in_specs=[
          pl.BlockSpec(memory_space=pltpu.SMEM),
          pl.BlockSpec((2, NR, 128), lambda: (0, 0, 0)),
          pl.BlockSpec((2, NR, 128), lambda: (0, 0, 0)),
      ],
      out_specs=[
          pl.BlockSpec((2, NR, 128), lambda: (0, 0, 0)),
          pl.BlockSpec((2, NR, 128), lambda: (0, 0, 0)),
      ],
      out_shape=[
          jax.ShapeDtypeStruct((2, NR, 128), jnp.float32),
          jax.ShapeDtypeStruct((2, NR, 128), jnp.float32),
      ],
  )(ks, msc, tanhv)


def _k6_body(kinv_ref, xm_ref, coef_ref, g_ref):
  b = pl.program_id(0)
  i = pl.program_id(1)
  coef = coef_ref[0]
  parts = [jnp.sum(xm_ref[0, c] * coef, axis=0, keepdims=True)
           for c in range(NCH)]
  res = jnp.broadcast_to(jnp.concatenate(parts, axis=-1) * kinv_ref[b],
                         (8, HID))

  @pl.when(i == 0)
  def _():
    g_ref[0] = res

  @pl.when(i > 0)
  def _():
    g_ref[0] += res


def _k6(kinv, xm, coef):
  return pl.pallas_call(
      _k6_body,
      grid=(2, NP // BN),
      in_specs=[
          pl.BlockSpec(memory_space=pltpu.SMEM),
          pl.BlockSpec((1, NCH, BN, CW), lambda b, i: (b, 0, i, 0)),
          pl.BlockSpec((1, BN, 1), lambda b, i: (b, i, 0)),
      ],
      out_specs=pl.BlockSpec((1, 8, HID), lambda b, i: (b, 0, 0)),
      out_shape=jax.ShapeDtypeStruct((2, 8, HID), jnp.float32),
  )(kinv, xm, coef)


def _ln_relu(x, g, b):
  mu = jnp.mean(x, axis=-1, keepdims=True)
  var = jnp.mean((x - mu) ** 2, axis=-1, keepdims=True)
  return jnp.maximum((x - mu) / jnp.sqrt(var + 1e-5) * g + b, 0.0)


def _dot16(a, b):
  return jnp.dot(a.astype(jnp.bfloat16), b.astype(jnp.bfloat16),
                 preferred_element_type=jnp.float32)


def _apply_fc(x, flat):
  ws, lns = flat[:8], flat[8:14]
  for i in range(3):
    x = _dot16(x, ws[2 * i][...]) + ws[2 * i + 1][...]
    x = _ln_relu(x, lns[2 * i][...], lns[2 * i + 1][...])
  return _dot16(x, ws[6][...]) + ws[7][...]


def _k7_body(*refs):
  g_ref = refs[0]
  out_ref = refs[-1]
  flat = refs[1:-1]
  o1 = _apply_fc(g_ref[0:1, :], flat[0:14])
  o2 = _apply_fc(g_ref[1:2, :], flat[14:28])
  out_ref[...] = _apply_fc(jnp.concatenate([o1, o2], axis=-1), flat[28:42])


def _zmap(nd):
  return lambda *a: (0,) * nd


def _k7(g, fc_gp, fc_sp, fc_fin):
  operands = [g]
  for fc in (fc_gp, fc_sp, fc_fin):
    for (W, b) in fc['lin']:
      operands += [W, b.reshape(1, -1)]
    for (ga, be) in fc['ln']:
      operands += [ga.reshape(1, -1), be.reshape(1, -1)]
  return pl.pallas_call(
      _k7_body,
      in_specs=[pl.BlockSpec(o.shape, _zmap(o.ndim)) for o in operands],
      out_specs=pl.BlockSpec((1, HID), lambda: (0, 0)),
      out_shape=jax.ShapeDtypeStruct((1, HID), jnp.float32),
  )(*operands)


def kernel(gp_x, gp_edge_index, sp_x, sp_edge_index, params):
  f32 = jnp.float32
  ratios = [0.8 * (0.8 ** i) for i in range(4)]

  loops = jnp.arange(N_NODES, dtype=jnp.int32)
  pad = (jnp.arange(EP - E_TOT, dtype=jnp.int32) % (NP - N_NODES)) + N_NODES

  def edges(ei):
    src = jnp.concatenate([ei[0].astype(jnp.int32), loops, pad])
    dst = jnp.concatenate([ei[1].astype(jnp.int32), loops, pad])
    return src.reshape(EW, 128), dst.reshape(EW, 128)

  gs, gd = edges(gp_edge_index)
  ss, sd = edges(sp_edge_index)
  srcw = jnp.stack([gs, ss])
  dstw = jnp.stack([gd, sd])

  def chunk(x):
    xp = jnp.zeros((NP, HID), f32).at[:N_NODES, :x.shape[1]].set(x)
    return xp.reshape(NP, NCH, CW).transpose(1, 0, 2)

  xm = jnp.stack([chunk(gp_x), chunk(sp_x)])
  alive0 = jnp.zeros((2, NP, 1), f32).at[:, :N_NODES].set(1.0)
  coef = alive0
  alive_nv = alive0

  n_live = N_NODES
  for i in range(4):
    Wg, bg = params['gp']['convs'][i]
    Ws, bs = params['sp']['convs'][i]
    W = jnp.stack([jnp.zeros((HID, HID), f32).at[:Wg.shape[0]].set(Wg),
                   jnp.zeros((HID, HID), f32).at[:Ws.shape[0]].set(Ws)])
    b = jnp.stack([bg, bs]).reshape(2, 1, HID)

    alive16 = jnp.pad(alive_nv, ((0, 0), (0, 0), (0, 15)))
    deg = _sc_deg(alive16, srcw, dstw)[:, :, 0:1]
    xw, y = _k1(xm, coef, deg, W)
    msg = _sc_wide(y.reshape(8, NP, CW), srcw, dstw).reshape(2, NCH, NP, CW)
    xm = _k3(msg, xw, deg, alive_nv, b)
    agg = _sc_wide(xm.reshape(8, NP, CW), srcw, dstw).reshape(2, NCH, NP, CW)

    pg = params['gp']['pools'][i]
    ps = params['sp']['pools'][i]
    relw = jnp.stack([pg[0].reshape(-1), ps[0].reshape(-1)]).reshape(2, 1, HID)
    rootw = jnp.stack([pg[2].reshape(-1),
                       ps[2].reshape(-1)]).reshape(2, 1, HID)
    relb = jnp.stack([pg[1].reshape(()), ps[1].reshape(())])

    msc, tanhv = _k4(agg, xm, relw, rootw, relb, alive_nv)
    k = int(math.ceil(ratios[i] * n_live))
    n_live = k
    ks = jnp.array([k, k], jnp.int32)
    coef, alive_nv = _k5(ks, msc.reshape(2, NR, 128),
                         tanhv.reshape(2, NR, 128))
    coef = coef.reshape(2, NP, 1)
    alive_nv = alive_nv.reshape(2, NP, 1)

  kinv = jnp.array([1.0 / n_live, 1.0 / n_live], f32)
  g = _k6(kinv, xm, coef)[:, 0, :]
  out = _k7(g, params['gp']['fc'], params['sp']['fc'], params['fc'])
  return out.reshape(HID)


# speedup vs baseline: 26.8314x; 2.5855x over previous
import math

import jax
import jax.numpy as jnp
from jax import lax
from jax.experimental import pallas as pl
from jax.experimental.pallas import tpu as pltpu
from jax.experimental.pallas import tpu_sc as plsc

N_NODES = 50000
N_EDGES = 800000
NP = 51200
NR = NP // 128
E_TOT = N_EDGES + N_NODES
EW = 6656
EP = EW * 128
HID = 128
NCH = 4
CW = 32
N_TILES = 16
WPT = EW // N_TILES
RPT = NP // N_TILES

BN = 2048
BNR = BN // 128

def _vec_mesh():
  return plsc.VectorSubcoreMesh(core_axis_name="c", subcore_axis_name="s",
                                num_cores=2, num_subcores=16)


EWP = EW + 4


def _db_pass(table_j, accum, srcw_b, dstw_b, w0, sidx, didx, rows,
             isem, gsem, ssem, K):
  nblk = WPT // K

  def idx_copies(s, blk):
    w = w0 + blk * K
    return [(srcw_b.at[w + k], sidx.at[s].at[k]) for k in range(K)] + \
           [(dstw_b.at[w + k], didx.at[s].at[k]) for k in range(K)]

  def issue_idx(s, blk):
    for a, bb in idx_copies(s, blk):
      pltpu.async_copy(a, bb, isem.at[s])

  def wait_idx(s, blk):
    for a, bb in idx_copies(s, blk):
      pltpu.make_async_copy(a, bb, isem.at[s]).wait()

  def g_copies(s):
    return [(table_j.at[sidx.at[s].at[k]],
             rows.at[s].at[pl.ds(k * 128, 128)]) for k in range(K)]

  def issue_gather(s):
    for a, bb in g_copies(s):
      pltpu.async_copy(a, bb, gsem.at[s])

  def wait_gather(s):
    for a, bb in g_copies(s):
      pltpu.make_async_copy(a, bb, gsem.at[s]).wait()

  def s_copies(s):
    return [(rows.at[s].at[pl.ds(k * 128, 128)],
             accum.at[didx.at[s].at[k]]) for k in range(K)]

  def issue_scatter(s):
    for a, bb in s_copies(s):
      pltpu.async_copy(a, bb, ssem.at[s], add=True)

  def wait_scatter(s):
    for a, bb in s_copies(s):
      pltpu.make_async_copy(a, bb, ssem.at[s]).wait()

  issue_idx(0, 0)
  issue_idx(1, 1)
  wait_idx(0, 0)
  issue_gather(0)

  @pl.loop(0, nblk // 2)
  def _(t):
    b0 = 2 * t
    wait_gather(0)
    issue_scatter(0)
    wait_idx(1, b0 + 1)
    issue_gather(1)
    wait_scatter(0)
    issue_idx(0, b0 + 2)
    wait_gather(1)
    issue_scatter(1)
    wait_idx(0, b0 + 2)
    issue_gather(0)
    wait_scatter(1)
    issue_idx(1, b0 + 3)

  wait_gather(0)
  wait_idx(1, nblk + 1)


def _sc_wide_body(table, srcw, dstw, out, accum, zbuf, sidx, didx, rows,
                  i1, i2, gs):
  c = lax.axis_index("c")
  s = lax.axis_index("s")
  w0 = s * WPT
  r0 = s * RPT

  @pl.loop(0, RPT // 32)
  def _(r):
    @pl.loop(0, CW, step=16)
    def _(col):
      zbuf[r, pl.ds(col, 16)] = jnp.zeros((16,), jnp.float32)

  for job in range(4):
    jid = c * 4 + job

    @pl.loop(0, 32)
    def _(z):
      pltpu.sync_copy(zbuf, accum.at[pl.ds(r0 + z * (RPT // 32), RPT // 32)])
    plsc.subcore_barrier()

    _db_pass(table.at[jid], accum, srcw.at[c], dstw.at[c], w0,
             sidx, didx, rows, i1, i2, gs, 2)

    plsc.subcore_barrier()
    pltpu.sync_copy(accum.at[pl.ds(r0, RPT)], out.at[jid].at[pl.ds(r0, RPT)])
    plsc.subcore_barrier()


@jax.jit
def _sc_wide(table, srcw, dstw):
  kern = pl.kernel(
      _sc_wide_body,
      out_type=jax.ShapeDtypeStruct((8, NP, CW), jnp.float32),
      mesh=_vec_mesh(),
      compiler_params=pltpu.CompilerParams(use_tc_tiling_on_sc=False),
      scratch_types=[
          pltpu.VMEM_SHARED((NP, CW), jnp.float32),
          pltpu.VMEM((RPT // 32, CW), jnp.float32),
          pltpu.VMEM((2, 2, 128), jnp.int32),
          pltpu.VMEM((2, 2, 128), jnp.int32),
          pltpu.VMEM((2, 256, CW), jnp.float32),
          pltpu.SemaphoreType.DMA((2,)),
          pltpu.SemaphoreType.DMA((2,)),
          pltpu.SemaphoreType.DMA((2,)),
      ],
  )
  return kern(table, srcw, dstw)


def _sc_deg_body(alive, srcw, dstw, out, accum, zbuf, sidx, didx, vals,
                 i1, i2, gs):
  b = lax.axis_index("c")
  s = lax.axis_index("s")
  w0 = s * WPT
  r0 = s * RPT

  @pl.loop(0, RPT // 8)
  def _(r):
    zbuf[r, pl.ds(0, 16)] = jnp.zeros((16,), jnp.float32)

  for z in range(8):
    pltpu.sync_copy(zbuf, accum.at[pl.ds(r0 + z * (RPT // 8), RPT // 8)])
  plsc.subcore_barrier()

  _db_pass(alive.at[b], accum, srcw.at[b], dstw.at[b], w0,
           sidx, didx, vals, i1, i2, gs, 4)

  plsc.subcore_barrier()
  pltpu.sync_copy(accum.at[pl.ds(r0, RPT)], out.at[b].at[pl.ds(r0, RPT)])
  plsc.subcore_barrier()


@jax.jit
def _sc_deg(alive, srcw, dstw):
  kern = pl.kernel(
      _sc_deg_body,
      out_type=jax.ShapeDtypeStruct((2, NP, 16), jnp.float32),
      mesh=_vec_mesh(),
      compiler_params=pltpu.CompilerParams(use_tc_tiling_on_sc=False),
      scratch_types=[
          pltpu.VMEM_SHARED((NP, 16), jnp.float32),
          pltpu.VMEM((RPT // 8, 16), jnp.float32),
          pltpu.VMEM((2, 4, 128), jnp.int32),
          pltpu.VMEM((2, 4, 128), jnp.int32),
          pltpu.VMEM((2, 512, 16), jnp.float32),
          pltpu.SemaphoreType.DMA((2,)),
          pltpu.SemaphoreType.DMA((2,)),
          pltpu.SemaphoreType.DMA((2,)),
      ],
  )
  return kern(alive, srcw, dstw)


def _k1_body(xm_ref, coef_ref, deg_ref, w_ref, xw_ref, y_ref):
  xm = jnp.concatenate([xm_ref[0, i] for i in range(NCH)], axis=-1)
  coef = coef_ref[0]
  xw = jnp.dot((xm * coef).astype(jnp.bfloat16), w_ref[0].astype(jnp.bfloat16),
               preferred_element_type=jnp.float32)
  xw_ref[0] = xw
  scale = 1.0 / jnp.sqrt(deg_ref[0] + 1.0)
  y = xw * scale
  for i in range(NCH):
    y_ref[0, i] = y[:, i * CW:(i + 1) * CW]


def _k1(xm, coef, deg, W):
  return pl.pallas_call(
      _k1_body,
      grid=(2, NP // BN),
      in_specs=[
          pl.BlockSpec((1, NCH, BN, CW), lambda b, i: (b, 0, i, 0)),
          pl.BlockSpec((1, BN, 1), lambda b, i: (b, i, 0)),
          pl.BlockSpec((1, BN, 1), lambda b, i: (b, i, 0)),
          pl.BlockSpec((1, HID, HID), lambda b, i: (b, 0, 0)),
      ],
      out_specs=[
          pl.BlockSpec((1, BN, HID), lambda b, i: (b, i, 0)),
          pl.BlockSpec((1, NCH, BN, CW), lambda b, i: (b, 0, i, 0)),
      ],
      out_shape=[
          jax.ShapeDtypeStruct((2, NP, HID), jnp.float32),
          jax.ShapeDtypeStruct((2, NCH, NP, CW), jnp.float32),
      ],
  )(xm, coef, deg, W)


def _k3_body(msg_ref, xw_ref, deg_ref, alive_ref, b_ref, out_ref):
  scale = 1.0 / jnp.sqrt(deg_ref[0] + 1.0)
  alive = alive_ref[0]
  msg = jnp.concatenate([msg_ref[0, i] for i in range(NCH)], axis=-1)
  xw = xw_ref[0]
  out = jnp.maximum(msg * scale + xw * (scale * scale) + b_ref[0], 0.0) * alive
  for i in range(NCH):
    out_ref[0, i] = out[:, i * CW:(i + 1) * CW]


def _k3(msg, xw, deg, alive, b):
  return pl.pallas_call(
      _k3_body,
      grid=(2, NP // BN),
      in_specs=[
          pl.BlockSpec((1, NCH, BN, CW), lambda b_, i: (b_, 0, i, 0)),
          pl.BlockSpec((1, BN, HID), lambda b_, i: (b_, i, 0)),
          pl.BlockSpec((1, BN, 1), lambda b_, i: (b_, i, 0)),
          pl.BlockSpec((1, BN, 1), lambda b_, i: (b_, i, 0)),
          pl.BlockSpec((1, 1, HID), lambda b_, i: (b_, 0, 0)),
      ],
      out_specs=pl.BlockSpec((1, NCH, BN, CW), lambda b_, i: (b_, 0, i, 0)),
      out_shape=jax.ShapeDtypeStruct((2, NCH, NP, CW), jnp.float32),
  )(msg, xw, deg, alive, b)


def _k4_body(agg_ref, xm_ref, relw_ref, rootw_ref, relb_ref, alive_ref,
             out_ref, tanh_ref):
  b = pl.program_id(0)
  def b16(v):
    return v.astype(jnp.bfloat16).astype(jnp.float32)

  srow = jnp.zeros((BN, 1), jnp.float32)
  for i in range(NCH):
    srow += jnp.sum(b16(agg_ref[0, i]) * b16(relw_ref[0, :, i * CW:(i + 1) * CW]),
                    axis=-1, keepdims=True)
    srow += jnp.sum(b16(xm_ref[0, i]) * b16(rootw_ref[0, :, i * CW:(i + 1) * CW]),
                    axis=-1, keepdims=True)
  pre = srow + relb_ref[b]
  alive = alive_ref[0]
  out_ref[0] = jnp.where(alive > 0.0, pre, -1e30)
  tanh_ref[0] = jnp.tanh(pre)


def _k4(agg, xm, relw, rootw, relb, alive):
  return pl.pallas_call(
      _k4_body,
      grid=(2, NP // BN),
      in_specs=[
          pl.BlockSpec((1, NCH, BN, CW), lambda b, i: (b, 0, i, 0)),
          pl.BlockSpec((1, NCH, BN, CW), lambda b, i: (b, 0, i, 0)),
          pl.BlockSpec((1, 1, HID), lambda b, i: (b, 0, 0)),
          pl.BlockSpec((1, 1, HID), lambda b, i: (b, 0, 0)),
          pl.BlockSpec(memory_space=pltpu.SMEM),
          pl.BlockSpec((1, BN, 1), lambda b, i: (b, i, 0)),
      ],
      out_specs=[
          pl.BlockSpec((1, BN, 1), lambda b, i: (b, i, 0)),
          pl.BlockSpec((1, BN, 1), lambda b, i: (b, i, 0)),
      ],
      out_shape=[
          jax.ShapeDtypeStruct((2, NP, 1), jnp.float32),
          jax.ShapeDtypeStruct((2, NP, 1), jnp.float32),
      ],
  )(agg, xm, relw, rootw, relb, alive)


def _k5_body(k_arr, msc_ref, tanh_ref, coef_ref, alive_ref):
  idx = (lax.broadcasted_iota(jnp.int32, (NR, 128), 0) * 128
         + lax.broadcasted_iota(jnp.int32, (NR, 128), 1))
  for b in range(2):
    k = k_arr[b]
    score = msc_ref[b]
    u = lax.bitcast_convert_type(score, jnp.int32)
    key = jnp.where(u >= 0, u, u ^ jnp.int32(0x7FFFFFFF))

    def cnt_ge(t):
      return jnp.sum((key >= t).astype(jnp.int32))

    t0 = jnp.where(cnt_ge(jnp.int32(0)) >= k, jnp.int32(0),
                   jnp.int32(-2147483648))

    def bit_step(i, t):
      t_try = jnp.bitwise_or(t, jnp.left_shift(jnp.int32(1), 30 - i))
      return jnp.where(cnt_ge(t_try) >= k, t_try, t)

    tau = lax.fori_loop(0, 31, bit_step, t0)
    n_gt = jnp.sum((key > tau).astype(jnp.int32))
    r = k - n_gt

    def tie_step(_, lohi):
      lo, hi = lohi
      mid = (lo + hi) // 2
      c = jnp.sum(((key == tau) & (idx < mid)).astype(jnp.int32))
      good = c >= r
      return jnp.where(good, lo, mid + 1), jnp.where(good, mid, hi)

    _, cut = lax.fori_loop(0, 17, tie_step, (jnp.int32(0), jnp.int32(NP)))
    sel = ((key > tau) | ((key == tau) & (idx < cut))).astype(jnp.float32)
    coef_ref[b] = tanh_ref[b] * sel
    alive_ref[b] = sel


def _k5(ks, msc, tanhv):
  return pl.pallas_call(
      _k5_body,
      in_specs=[
          pl.BlockSpec(memory_space=pltpu.SMEM),
          pl.BlockSpec((2, NR, 128), lambda: (0, 0, 0)),
          pl.BlockSpec((2, NR, 128), lambda: (0, 0, 0)),
      ],
      out_specs=[
          pl.BlockSpec((2, NR, 128), lambda: (0, 0, 0)),
          pl.BlockSpec((2, NR, 128), lambda: (0, 0, 0)),
      ],
      out_shape=[
          jax.ShapeDtypeStruct((2, NR, 128), jnp.float32),
          jax.ShapeDtypeStruct((2, NR, 128), jnp.float32),
      ],
  )(ks, msc, tanhv)


def _k6_body(kinv_ref, xm_ref, coef_ref, g_ref):
  b = pl.program_id(0)
  i = pl.program_id(1)
  coef = coef_ref[0]
  parts = [jnp.sum(xm_ref[0, c] * coef, axis=0, keepdims=True)
           for c in range(NCH)]
  res = jnp.broadcast_to(jnp.concatenate(parts, axis=-1) * kinv_ref[b],
                         (8, HID))

  @pl.when(i == 0)
  def _():
    g_ref[0] = res

  @pl.when(i > 0)
  def _():
    g_ref[0] += res


def _k6(kinv, xm, coef):
  return pl.pallas_call(
      _k6_body,
      grid=(2, NP // BN),
      in_specs=[
          pl.BlockSpec(memory_space=pltpu.SMEM),
          pl.BlockSpec((1, NCH, BN, CW), lambda b, i: (b, 0, i, 0)),
          pl.BlockSpec((1, BN, 1), lambda b, i: (b, i, 0)),
      ],
      out_specs=pl.BlockSpec((1, 8, HID), lambda b, i: (b, 0, 0)),
      out_shape=jax.ShapeDtypeStruct((2, 8, HID), jnp.float32),
  )(kinv, xm, coef)


def _ln_relu(x, g, b):
  mu = jnp.mean(x, axis=-1, keepdims=True)
  var = jnp.mean((x - mu) ** 2, axis=-1, keepdims=True)
  return jnp.maximum((x - mu) / jnp.sqrt(var + 1e-5) * g + b, 0.0)


def _dot16(a, b):
  return jnp.dot(a.astype(jnp.bfloat16), b.astype(jnp.bfloat16),
                 preferred_element_type=jnp.float32)


def _apply_fc(x, flat):
  ws, lns = flat[:8], flat[8:14]
  for i in range(3):
    x = _dot16(x, ws[2 * i][...]) + ws[2 * i + 1][...]
    x = _ln_relu(x, lns[2 * i][...], lns[2 * i + 1][...])
  return _dot16(x, ws[6][...]) + ws[7][...]


def _k7_body(*refs):
  g_ref = refs[0]
  out_ref = refs[-1]
  flat = refs[1:-1]
  o1 = _apply_fc(g_ref[0:1, :], flat[0:14])
  o2 = _apply_fc(g_ref[1:2, :], flat[14:28])
  out_ref[...] = _apply_fc(jnp.concatenate([o1, o2], axis=-1), flat[28:42])


def _zmap(nd):
  return lambda *a: (0,) * nd


def _k7(g, fc_gp, fc_sp, fc_fin):
  operands = [g]
  for fc in (fc_gp, fc_sp, fc_fin):
    for (W, b) in fc['lin']:
      operands += [W, b.reshape(1, -1)]
    for (ga, be) in fc['ln']:
      operands += [ga.reshape(1, -1), be.reshape(1, -1)]
  return pl.pallas_call(
      _k7_body,
      in_specs=[pl.BlockSpec(o.shape, _zmap(o.ndim)) for o in operands],
      out_specs=pl.BlockSpec((1, HID), lambda: (0, 0)),
      out_shape=jax.ShapeDtypeStruct((1, HID), jnp.float32),
  )(*operands)


def kernel(gp_x, gp_edge_index, sp_x, sp_edge_index, params):
  f32 = jnp.float32
  ratios = [0.8 * (0.8 ** i) for i in range(4)]

  loops = jnp.arange(N_NODES, dtype=jnp.int32)
  pad = (jnp.arange(EWP * 128 - E_TOT, dtype=jnp.int32)
         % (NP - N_NODES)) + N_NODES

  def edges(ei):
    src = jnp.concatenate([ei[0].astype(jnp.int32), loops, pad])
    dst = jnp.concatenate([ei[1].astype(jnp.int32), loops, pad])
    return src.reshape(EWP, 128), dst.reshape(EWP, 128)

  gs, gd = edges(gp_edge_index)
  ss, sd = edges(sp_edge_index)
  srcw = jnp.stack([gs, ss])
  dstw = jnp.stack([gd, sd])

  def chunk(x):
    xp = jnp.zeros((NP, HID), f32).at[:N_NODES, :x.shape[1]].set(x)
    return xp.reshape(NP, NCH, CW).transpose(1, 0, 2)

  xm = jnp.stack([chunk(gp_x), chunk(sp_x)])
  alive0 = jnp.zeros((2, NP, 1), f32).at[:, :N_NODES].set(1.0)
  coef = alive0
  alive_nv = alive0

  n_live = N_NODES
  for i in range(4):
    Wg, bg = params['gp']['convs'][i]
    Ws, bs = params['sp']['convs'][i]
    W = jnp.stack([jnp.zeros((HID, HID), f32).at[:Wg.shape[0]].set(Wg),
                   jnp.zeros((HID, HID), f32).at[:Ws.shape[0]].set(Ws)])
    b = jnp.stack([bg, bs]).reshape(2, 1, HID)

    alive16 = jnp.pad(alive_nv, ((0, 0), (0, 0), (0, 15)))
    deg = _sc_deg(alive16, srcw, dstw)[:, :, 0:1]
    xw, y = _k1(xm, coef, deg, W)
    msg = _sc_wide(y.reshape(8, NP, CW), srcw, dstw).reshape(2, NCH, NP, CW)
    xm = _k3(msg, xw, deg, alive_nv, b)
    agg = _sc_wide(xm.reshape(8, NP, CW), srcw, dstw).reshape(2, NCH, NP, CW)

    pg = params['gp']['pools'][i]
    ps = params['sp']['pools'][i]
    relw = jnp.stack([pg[0].reshape(-1), ps[0].reshape(-1)]).reshape(2, 1, HID)
    rootw = jnp.stack([pg[2].reshape(-1),
                       ps[2].reshape(-1)]).reshape(2, 1, HID)
    relb = jnp.stack([pg[1].reshape(()), ps[1].reshape(())])

    msc, tanhv = _k4(agg, xm, relw, rootw, relb, alive_nv)
    k = int(math.ceil(ratios[i] * n_live))
    n_live = k
    ks = jnp.array([k, k], jnp.int32)
    coef, alive_nv = _k5(ks, msc.reshape(2, NR, 128),
                         tanhv.reshape(2, NR, 128))
    coef = coef.reshape(2, NP, 1)
    alive_nv = alive_nv.reshape(2, NP, 1)

  kinv = jnp.array([1.0 / n_live, 1.0 / n_live], f32)
  g = _k6(kinv, xm, coef)[:, 0, :]
  out = _k7(g, params['gp']['fc'], params['sp']['fc'], params['fc'])
  return out.reshape(HID)


# speedup vs baseline: 27.9966x; 1.0434x over previous
import math

import jax
import jax.numpy as jnp
from jax import lax
from jax.experimental import pallas as pl
from jax.experimental.pallas import tpu as pltpu
from jax.experimental.pallas import tpu_sc as plsc

N_NODES = 50000
N_EDGES = 800000
NP = 51200
NR = NP // 128
E_TOT = N_EDGES + N_NODES
EW = 6656
EP = EW * 128
HID = 128
NCH = 4
CW = 32
N_TILES = 16
WPT = EW // N_TILES
RPT = NP // N_TILES

BN = 2048
BNR = BN // 128

def _vec_mesh():
  return plsc.VectorSubcoreMesh(core_axis_name="c", subcore_axis_name="s",
                                num_cores=2, num_subcores=16)


EWP = EW + 12


def _db_pass(table_j, accum, srcw_b, dstw_b, w0, sidx, didx, rows,
             isem, gsem, ssem, K):
  nblk = WPT // K

  def _idx_pairs(iset, blk):
    w = w0 + blk * K
    return [(srcw_b.at[w + k], sidx.at[iset].at[k]) for k in range(K)] + \
           [(dstw_b.at[w + k], didx.at[iset].at[k]) for k in range(K)]

  def issue_idx(iset, blk):
    for a, bb in _idx_pairs(iset, blk):
      pltpu.async_copy(a, bb, isem.at[iset])

  def wait_idx(iset, blk):
    for a, bb in _idx_pairs(iset, blk):
      pltpu.make_async_copy(a, bb, isem.at[iset]).wait()

  def g_copies(rset, iset):
    return [(table_j.at[sidx.at[iset].at[k]],
             rows.at[rset].at[pl.ds(k * 128, 128)]) for k in range(K)]

  def issue_gather(rset, iset):
    for a, bb in g_copies(rset, iset):
      pltpu.async_copy(a, bb, gsem.at[rset])

  def wait_gather(rset, iset):
    for a, bb in g_copies(rset, iset):
      pltpu.make_async_copy(a, bb, gsem.at[rset]).wait()

  def s_copies(rset, iset):
    return [(rows.at[rset].at[pl.ds(k * 128, 128)],
             accum.at[didx.at[iset].at[k]]) for k in range(K)]

  def issue_scatter(rset, iset):
    for a, bb in s_copies(rset, iset):
      pltpu.async_copy(a, bb, ssem.at[rset], add=True)

  def wait_scatter(rset, iset):
    for a, bb in s_copies(rset, iset):
      pltpu.make_async_copy(a, bb, ssem.at[rset]).wait()

  issue_idx(0, 0)
  issue_idx(1, 1)
  issue_idx(2, 2)
  wait_idx(0, 0)
  issue_gather(0, 0)

  def sub(b0, t, A, B, C, D, first_guard):
    wait_gather(0, A)
    issue_scatter(0, A)
    wait_idx(B, b0 + 1)
    if first_guard:
      @pl.when(t > 0)
      def _():
        wait_scatter(1, D)
    else:
      wait_scatter(1, D)
    issue_gather(1, B)
    issue_idx(D, b0 + 3)
    wait_scatter(0, A)
    wait_idx(C, b0 + 2)
    issue_gather(0, C)
    issue_idx(A, b0 + 4)
    wait_gather(1, B)
    issue_scatter(1, B)

  @pl.loop(0, nblk // 4)
  def _(t):
    b0 = 4 * t
    sub(b0, t, 0, 1, 2, 3, True)
    sub(b0 + 2, t, 2, 3, 0, 1, False)

  wait_scatter(1, 3)
  wait_gather(0, 0)
  wait_idx(1, nblk + 1)
  wait_idx(2, nblk + 2)


def _sc_wide_body(table, srcw, dstw, out, accum, zbuf, sidx, didx, rows,
                  i1, i2, gs):
  c = lax.axis_index("c")
  s = lax.axis_index("s")
  w0 = s * WPT
  r0 = s * RPT

  @pl.loop(0, RPT // 32)
  def _(r):
    @pl.loop(0, CW, step=16)
    def _(col):
      zbuf[r, pl.ds(col, 16)] = jnp.zeros((16,), jnp.float32)

  for job in range(4):
    jid = c * 4 + job

    @pl.loop(0, 32)
    def _(z):
      pltpu.sync_copy(zbuf, accum.at[pl.ds(r0 + z * (RPT // 32), RPT // 32)])
    plsc.subcore_barrier()

    _db_pass(table.at[jid], accum, srcw.at[c], dstw.at[c], w0,
             sidx, didx, rows, i1, i2, gs, 2)

    plsc.subcore_barrier()
    pltpu.sync_copy(accum.at[pl.ds(r0, RPT)], out.at[jid].at[pl.ds(r0, RPT)])
    plsc.subcore_barrier()


@jax.jit
def _sc_wide(table, srcw, dstw):
  kern = pl.kernel(
      _sc_wide_body,
      out_type=jax.ShapeDtypeStruct((8, NP, CW), jnp.float32),
      mesh=_vec_mesh(),
      compiler_params=pltpu.CompilerParams(use_tc_tiling_on_sc=False),
      scratch_types=[
          pltpu.VMEM_SHARED((NP, CW), jnp.float32),
          pltpu.VMEM((RPT // 32, CW), jnp.float32),
          pltpu.VMEM((4, 2, 128), jnp.int32),
          pltpu.VMEM((4, 2, 128), jnp.int32),
          pltpu.VMEM((2, 256, CW), jnp.float32),
          pltpu.SemaphoreType.DMA((4,)),
          pltpu.SemaphoreType.DMA((2,)),
          pltpu.SemaphoreType.DMA((2,)),
      ],
  )
  return kern(table, srcw, dstw)


def _sc_deg_body(alive, srcw, dstw, out, accum, zbuf, sidx, didx, vals,
                 i1, i2, gs):
  b = lax.axis_index("c")
  s = lax.axis_index("s")
  w0 = s * WPT
  r0 = s * RPT

  @pl.loop(0, RPT // 8)
  def _(r):
    zbuf[r, pl.ds(0, 16)] = jnp.zeros((16,), jnp.float32)

  for z in range(8):
    pltpu.sync_copy(zbuf, accum.at[pl.ds(r0 + z * (RPT // 8), RPT // 8)])
  plsc.subcore_barrier()

  _db_pass(alive.at[b], accum, srcw.at[b], dstw.at[b], w0,
           sidx, didx, vals, i1, i2, gs, 4)

  plsc.subcore_barrier()
  pltpu.sync_copy(accum.at[pl.ds(r0, RPT)], out.at[b].at[pl.ds(r0, RPT)])
  plsc.subcore_barrier()


@jax.jit
def _sc_deg(alive, srcw, dstw):
  kern = pl.kernel(
      _sc_deg_body,
      out_type=jax.ShapeDtypeStruct((2, NP, 16), jnp.float32),
      mesh=_vec_mesh(),
      compiler_params=pltpu.CompilerParams(use_tc_tiling_on_sc=False),
      scratch_types=[
          pltpu.VMEM_SHARED((NP, 16), jnp.float32),
          pltpu.VMEM((RPT // 8, 16), jnp.float32),
          pltpu.VMEM((4, 4, 128), jnp.int32),
          pltpu.VMEM((4, 4, 128), jnp.int32),
          pltpu.VMEM((2, 512, 16), jnp.float32),
          pltpu.SemaphoreType.DMA((4,)),
          pltpu.SemaphoreType.DMA((2,)),
          pltpu.SemaphoreType.DMA((2,)),
      ],
  )
  return kern(alive, srcw, dstw)


def _k1_body(xm_ref, coef_ref, deg_ref, w_ref, xw_ref, y_ref):
  xm = jnp.concatenate([xm_ref[0, i] for i in range(NCH)], axis=-1)
  coef = coef_ref[0]
  xw = jnp.dot((xm * coef).astype(jnp.bfloat16), w_ref[0].astype(jnp.bfloat16),
               preferred_element_type=jnp.float32)
  xw_ref[0] = xw
  scale = 1.0 / jnp.sqrt(deg_ref[0] + 1.0)
  y = xw * scale
  for i in range(NCH):
    y_ref[0, i] = y[:, i * CW:(i + 1) * CW]


def _k1(xm, coef, deg, W):
  return pl.pallas_call(
      _k1_body,
      grid=(2, NP // BN),
      in_specs=[
          pl.BlockSpec((1, NCH, BN, CW), lambda b, i: (b, 0, i, 0)),
          pl.BlockSpec((1, BN, 1), lambda b, i: (b, i, 0)),
          pl.BlockSpec((1, BN, 1), lambda b, i: (b, i, 0)),
          pl.BlockSpec((1, HID, HID), lambda b, i: (b, 0, 0)),
      ],
      out_specs=[
          pl.BlockSpec((1, BN, HID), lambda b, i: (b, i, 0)),
          pl.BlockSpec((1, NCH, BN, CW), lambda b, i: (b, 0, i, 0)),
      ],
      out_shape=[
          jax.ShapeDtypeStruct((2, NP, HID), jnp.float32),
          jax.ShapeDtypeStruct((2, NCH, NP, CW), jnp.float32),
      ],
  )(xm, coef, deg, W)


def _k3_body(msg_ref, xw_ref, deg_ref, alive_ref, b_ref, out_ref):
  scale = 1.0 / jnp.sqrt(deg_ref[0] + 1.0)
  alive = alive_ref[0]
  msg = jnp.concatenate([msg_ref[0, i] for i in range(NCH)], axis=-1)
  xw = xw_ref[0]
  out = jnp.maximum(msg * scale + xw * (scale * scale) + b_ref[0], 0.0) * alive
  for i in range(NCH):
    out_ref[0, i] = out[:, i * CW:(i + 1) * CW]


def _k3(msg, xw, deg, alive, b):
  return pl.pallas_call(
      _k3_body,
      grid=(2, NP // BN),
      in_specs=[
          pl.BlockSpec((1, NCH, BN, CW), lambda b_, i: (b_, 0, i, 0)),
          pl.BlockSpec((1, BN, HID), lambda b_, i: (b_, i, 0)),
          pl.BlockSpec((1, BN, 1), lambda b_, i: (b_, i, 0)),
          pl.BlockSpec((1, BN, 1), lambda b_, i: (b_, i, 0)),
          pl.BlockSpec((1, 1, HID), lambda b_, i: (b_, 0, 0)),
      ],
      out_specs=pl.BlockSpec((1, NCH, BN, CW), lambda b_, i: (b_, 0, i, 0)),
      out_shape=jax.ShapeDtypeStruct((2, NCH, NP, CW), jnp.float32),
  )(msg, xw, deg, alive, b)


def _k4_body(agg_ref, xm_ref, relw_ref, rootw_ref, relb_ref, alive_ref,
             out_ref, tanh_ref):
  b = pl.program_id(0)
  def b16(v):
    return v.astype(jnp.bfloat16).astype(jnp.float32)

  srow = jnp.zeros((BN, 1), jnp.float32)
  for i in range(NCH):
    srow += jnp.sum(b16(agg_ref[0, i]) * b16(relw_ref[0, :, i * CW:(i + 1) * CW]),
                    axis=-1, keepdims=True)
    srow += jnp.sum(b16(xm_ref[0, i]) * b16(rootw_ref[0, :, i * CW:(i + 1) * CW]),
                    axis=-1, keepdims=True)
  pre = srow + relb_ref[b]
  alive = alive_ref[0]
  out_ref[0] = jnp.where(alive > 0.0, pre, -1e30)
  tanh_ref[0] = jnp.tanh(pre)


def _k4(agg, xm, relw, rootw, relb, alive):
  return pl.pallas_call(
      _k4_body,
      grid=(2, NP // BN),
      in_specs=[
          pl.BlockSpec((1, NCH, BN, CW), lambda b, i: (b, 0, i, 0)),
          pl.BlockSpec((1, NCH, BN, CW), lambda b, i: (b, 0, i, 0)),
          pl.BlockSpec((1, 1, HID), lambda b, i: (b, 0, 0)),
          pl.BlockSpec((1, 1, HID), lambda b, i: (b, 0, 0)),
          pl.BlockSpec(memory_space=pltpu.SMEM),
          pl.BlockSpec((1, BN, 1), lambda b, i: (b, i, 0)),
      ],
      out_specs=[
          pl.BlockSpec((1, BN, 1), lambda b, i: (b, i, 0)),
          pl.BlockSpec((1, BN, 1), lambda b, i: (b, i, 0)),
      ],
      out_shape=[
          jax.ShapeDtypeStruct((2, NP, 1), jnp.float32),
          jax.ShapeDtypeStruct((2, NP, 1), jnp.float32),
      ],
  )(agg, xm, relw, rootw, relb, alive)


def _k5_body(k_arr, msc_ref, tanh_ref, coef_ref, alive_ref):
  idx = (lax.broadcasted_iota(jnp.int32, (NR, 128), 0) * 128
         + lax.broadcasted_iota(jnp.int32, (NR, 128), 1))
  for b in range(2):
    k = k_arr[b]
    score = msc_ref[b]
    u = lax.bitcast_convert_type(score, jnp.int32)
    key = jnp.where(u >= 0, u, u ^ jnp.int32(0x7FFFFFFF))

    def cnt_ge(t):
      return jnp.sum((key >= t).astype(jnp.int32))

    t0 = jnp.where(cnt_ge(jnp.int32(0)) >= k, jnp.int32(0),
                   jnp.int32(-2147483648))

    def bit_step(i, t):
      t_try = jnp.bitwise_or(t, jnp.left_shift(jnp.int32(1), 30 - i))
      return jnp.where(cnt_ge(t_try) >= k, t_try, t)

    tau = lax.fori_loop(0, 31, bit_step, t0)
    n_gt = jnp.sum((key > tau).astype(jnp.int32))
    r = k - n_gt

    def tie_step(_, lohi):
      lo, hi = lohi
      mid = (lo + hi) // 2
      c = jnp.sum(((key == tau) & (idx < mid)).astype(jnp.int32))
      good = c >= r
      return jnp.where(good, lo, mid + 1), jnp.where(good, mid, hi)

    _, cut = lax.fori_loop(0, 17, tie_step, (jnp.int32(0), jnp.int32(NP)))
    sel = ((key > tau) | ((key == tau) & (idx < cut))).astype(jnp.float32)
    coef_ref[b] = tanh_ref[b] * sel
    alive_ref[b] = sel


def _k5(ks, msc, tanhv):
  return pl.pallas_call(
      _k5_body,
      in_specs=[
          pl.BlockSpec(memory_space=pltpu.SMEM),
          pl.BlockSpec((2, NR, 128), lambda: (0, 0, 0)),
          pl.BlockSpec((2, NR, 128), lambda: (0, 0, 0)),
      ],
      out_specs=[
          pl.BlockSpec((2, NR, 128), lambda: (0, 0, 0)),
          pl.BlockSpec((2, NR, 128), lambda: (0, 0, 0)),
      ],
      out_shape=[
          jax.ShapeDtypeStruct((2, NR, 128), jnp.float32),
          jax.ShapeDtypeStruct((2, NR, 128), jnp.float32),
      ],
  )(ks, msc, tanhv)


def _k6_body(kinv_ref, xm_ref, coef_ref, g_ref):
  b = pl.program_id(0)
  i = pl.program_id(1)
  coef = coef_ref[0]
  parts = [jnp.sum(xm_ref[0, c] * coef, axis=0, keepdims=True)
           for c in range(NCH)]
  res = jnp.broadcast_to(jnp.concatenate(parts, axis=-1) * kinv_ref[b],
                         (8, HID))

  @pl.when(i == 0)
  def _():
    g_ref[0] = res

  @pl.when(i > 0)
  def _():
    g_ref[0] += res


def _k6(kinv, xm, coef):
  return pl.pallas_call(
      _k6_body,
      grid=(2, NP // BN),
      in_specs=[
          pl.BlockSpec(memory_space=pltpu.SMEM),
          pl.BlockSpec((1, NCH, BN, CW), lambda b, i: (b, 0, i, 0)),
          pl.BlockSpec((1, BN, 1), lambda b, i: (b, i, 0)),
      ],
      out_specs=pl.BlockSpec((1, 8, HID), lambda b, i: (b, 0, 0)),
      out_shape=jax.ShapeDtypeStruct((2, 8, HID), jnp.float32),
  )(kinv, xm, coef)


def _ln_relu(x, g, b):
  mu = jnp.mean(x, axis=-1, keepdims=True)
  var = jnp.mean((x - mu) ** 2, axis=-1, keepdims=True)
  return jnp.maximum((x - mu) / jnp.sqrt(var + 1e-5) * g + b, 0.0)


def _dot16(a, b):
  return jnp.dot(a.astype(jnp.bfloat16), b.astype(jnp.bfloat16),
                 preferred_element_type=jnp.float32)


def _apply_fc(x, flat):
  ws, lns = flat[:8], flat[8:14]
  for i in range(3):
    x = _dot16(x, ws[2 * i][...]) + ws[2 * i + 1][...]
    x = _ln_relu(x, lns[2 * i][...], lns[2 * i + 1][...])
  return _dot16(x, ws[6][...]) + ws[7][...]


def _k7_body(*refs):
  g_ref = refs[0]
  out_ref = refs[-1]
  flat = refs[1:-1]
  o1 = _apply_fc(g_ref[0:1, :], flat[0:14])
  o2 = _apply_fc(g_ref[1:2, :], flat[14:28])
  out_ref[...] = _apply_fc(jnp.concatenate([o1, o2], axis=-1), flat[28:42])


def _zmap(nd):
  return lambda *a: (0,) * nd


def _k7(g, fc_gp, fc_sp, fc_fin):
  operands = [g]
  for fc in (fc_gp, fc_sp, fc_fin):
    for (W, b) in fc['lin']:
      operands += [W, b.reshape(1, -1)]
    for (ga, be) in fc['ln']:
      operands += [ga.reshape(1, -1), be.reshape(1, -1)]
  return pl.pallas_call(
      _k7_body,
      in_specs=[pl.BlockSpec(o.shape, _zmap(o.ndim)) for o in operands],
      out_specs=pl.BlockSpec((1, HID), lambda: (0, 0)),
      out_shape=jax.ShapeDtypeStruct((1, HID), jnp.float32),
  )(*operands)


def kernel(gp_x, gp_edge_index, sp_x, sp_edge_index, params):
  f32 = jnp.float32
  ratios = [0.8 * (0.8 ** i) for i in range(4)]

  loops = jnp.arange(N_NODES, dtype=jnp.int32)
  pad = (jnp.arange(EWP * 128 - E_TOT, dtype=jnp.int32)
         % (NP - N_NODES)) + N_NODES

  def edges(ei):
    src = jnp.concatenate([ei[0].astype(jnp.int32), loops, pad])
    dst = jnp.concatenate([ei[1].astype(jnp.int32), loops, pad])
    return src.reshape(EWP, 128), dst.reshape(EWP, 128)

  gs, gd = edges(gp_edge_index)
  ss, sd = edges(sp_edge_index)
  srcw = jnp.stack([gs, ss])
  dstw = jnp.stack([gd, sd])

  def chunk(x):
    xp = jnp.zeros((NP, HID), f32).at[:N_NODES, :x.shape[1]].set(x)
    return xp.reshape(NP, NCH, CW).transpose(1, 0, 2)

  xm = jnp.stack([chunk(gp_x), chunk(sp_x)])
  alive0 = jnp.zeros((2, NP, 1), f32).at[:, :N_NODES].set(1.0)
  coef = alive0
  alive_nv = alive0

  n_live = N_NODES
  for i in range(4):
    Wg, bg = params['gp']['convs'][i]
    Ws, bs = params['sp']['convs'][i]
    W = jnp.stack([jnp.zeros((HID, HID), f32).at[:Wg.shape[0]].set(Wg),
                   jnp.zeros((HID, HID), f32).at[:Ws.shape[0]].set(Ws)])
    b = jnp.stack([bg, bs]).reshape(2, 1, HID)

    alive16 = jnp.pad(alive_nv, ((0, 0), (0, 0), (0, 15)))
    deg = _sc_deg(alive16, srcw, dstw)[:, :, 0:1]
    xw, y = _k1(xm, coef, deg, W)
    msg = _sc_wide(y.reshape(8, NP, CW), srcw, dstw).reshape(2, NCH, NP, CW)
    xm = _k3(msg, xw, deg, alive_nv, b)
    agg = _sc_wide(xm.reshape(8, NP, CW), srcw, dstw).reshape(2, NCH, NP, CW)

    pg = params['gp']['pools'][i]
    ps = params['sp']['pools'][i]
    relw = jnp.stack([pg[0].reshape(-1), ps[0].reshape(-1)]).reshape(2, 1, HID)
    rootw = jnp.stack([pg[2].reshape(-1),
                       ps[2].reshape(-1)]).reshape(2, 1, HID)
    relb = jnp.stack([pg[1].reshape(()), ps[1].reshape(())])

    msc, tanhv = _k4(agg, xm, relw, rootw, relb, alive_nv)
    k = int(math.ceil(ratios[i] * n_live))
    n_live = k
    ks = jnp.array([k, k], jnp.int32)
    coef, alive_nv = _k5(ks, msc.reshape(2, NR, 128),
                         tanhv.reshape(2, NR, 128))
    coef = coef.reshape(2, NP, 1)
    alive_nv = alive_nv.reshape(2, NP, 1)

  kinv = jnp.array([1.0 / n_live, 1.0 / n_live], f32)
  g = _k6(kinv, xm, coef)[:, 0, :]
  out = _k7(g, params['gp']['fc'], params['sp']['fc'], params['fc'])
  return out.reshape(HID)
